# Initial kernel scaffold; baseline (speedup 1.0000x reference)
import jax
import jax.numpy as jnp
from jax.experimental import pallas as pl


def kernel(x, emb_weight):
    raise NotImplementedError("write your pallas kernel here")


# speedup vs baseline: 2.2363x; 2.2363x over previous
import functools

import jax
import jax.numpy as jnp
from jax import lax
from jax.experimental import pallas as pl
from jax.experimental.pallas import tpu as pltpu
from jax.experimental.pallas import tpu_sc as plsc

_NUM_CODES = 8192
_DIM = 32
_COMMIT = 0.25
_TB = 256
_CH = 1024
_NW = 32
_BW = 256


def _vq_body(f_ref, f2_ref, et_ref, e2_ref, dist_ref, enc_ref, idx_ref):
    f = f_ref[...]
    f2 = f2_ref[...]
    mm = lax.dot_general(f, et_ref[...], (((1,), (0,)), ((), ())),
                         preferred_element_type=jnp.float32)

    ids = lax.broadcasted_iota(jnp.int32, (_TB, _CH), 1)
    run_min = None
    run_idx = None
    for c in range(_NUM_CODES // _CH):
        sl = pl.ds(c * _CH, _CH)
        dc = (f2 + e2_ref[:, sl]) - 2.0 * mm[:, c * _CH:(c + 1) * _CH]
        dist_ref[:, sl] = dc
        mc = jnp.min(dc, axis=1, keepdims=True)
        idc = jnp.min(jnp.where(dc == mc, ids, _NUM_CODES), axis=1,
                      keepdims=True) + c * _CH
        if c == 0:
            run_min, run_idx = mc, idc
        else:
            better = mc < run_min
            run_idx = jnp.where(better, idc, run_idx)
            run_min = jnp.where(better, mc, run_min)
    idx_ref[...] = run_idx

    for c in range(_NUM_CODES // _CH):
        enc_ref[:, pl.ds(c * _CH, _CH)] = (
            ids == run_idx - c * _CH).astype(jnp.float32)


def _sc_stage_body(flat_hbm, ep_hbm, idx_hbm, qst_hbm, sse_hbm, hist_hbm,
                   idx_v, rows_v, f_v, qst_v, acc_v, lh_v, ones_v, shared,
                   sem):
    sid = lax.axis_index("s")
    cid = lax.axis_index("c")
    wid = sid * 2 + cid
    base = wid * _BW
    pltpu.sync_copy(idx_hbm.at[pl.ds(base, _BW)], idx_v)
    pltpu.async_copy(ep_hbm.at[idx_v], rows_v, sem).wait()
    pltpu.sync_copy(flat_hbm.at[pl.ds(base, _BW)], f_v)

    zero = jnp.zeros((16,), jnp.float32)
    ones16 = jnp.ones((16,), jnp.float32)

    def ew_body(r, acc):
        for c in (0, 16):
            qv = rows_v[r, pl.ds(c, 16)]
            fv = f_v[r, pl.ds(c, 16)]
            qst_v[r, pl.ds(c, 16)] = fv + (qv - fv)
            d = qv - fv
            acc = acc + d * d
        return acc

    acc = lax.fori_loop(0, _BW, ew_body, zero)
    acc_v[...] = acc
    pltpu.sync_copy(qst_v, qst_hbm.at[pl.ds(base, _BW)])
    pltpu.sync_copy(acc_v, sse_hbm.at[wid])

    def z_body(k, carry):
        lh_v[pl.ds(k * 16, 16)] = zero
        return carry

    lax.fori_loop(0, _NUM_CODES // 16, z_body, 0)

    def o_body(k, carry):
        ones_v[pl.ds(k * 16, 16)] = ones16
        return carry

    lax.fori_loop(0, _BW // 16, o_body, 0)

    @pl.when(sid == 0)
    def _():
        pltpu.sync_copy(lh_v, shared)

    plsc.subcore_barrier()
    pltpu.sync_copy(ones_v, shared.at[idx_v], add=True)
    plsc.subcore_barrier()

    @pl.when(sid == 0)
    def _():
        pltpu.sync_copy(shared, lh_v)
        pltpu.sync_copy(lh_v, hist_hbm.at[cid])


def _finalize_body(n_total, sse_ref, hist_ref, loss_ref, perp_ref):
    mse = jnp.sum(sse_ref[...]) / n_total
    loss_ref[...] = jnp.reshape(mse + _COMMIT * mse, (1, 1))
    hist = jnp.sum(hist_ref[...], axis=0, keepdims=True)
    p = hist * (1.0 / n_total * _DIM)
    ent = jnp.sum(p * jnp.log(p + 1e-10))
    perp_ref[...] = jnp.reshape(jnp.exp(-ent), (1, 1))


def kernel(x, emb_weight):
    inputs = jnp.transpose(x, (0, 2, 3, 1))
    inputs_shape = inputs.shape
    flat = inputs.reshape(-1, _DIM)
    n_tok = flat.shape[0]
    nsteps = n_tok // _TB
    n_total = n_tok * _DIM

    flat_sq = jnp.sum(flat ** 2, axis=1, keepdims=True)
    emb_sq = jnp.sum(emb_weight ** 2, axis=1)[None, :]
    emb_t = emb_weight.T

    dist, enc, idx = pl.pallas_call(
        _vq_body,
        grid=(nsteps,),
        in_specs=[
            pl.BlockSpec((_TB, _DIM), lambda i: (i, 0)),
            pl.BlockSpec((_TB, 1), lambda i: (i, 0)),
            pl.BlockSpec((_DIM, _NUM_CODES), lambda i: (0, 0)),
            pl.BlockSpec((1, _NUM_CODES), lambda i: (0, 0)),
        ],
        out_specs=[
            pl.BlockSpec((_TB, _NUM_CODES), lambda i: (i, 0)),
            pl.BlockSpec((_TB, _NUM_CODES), lambda i: (i, 0)),
            pl.BlockSpec((_TB, 1), lambda i: (i, 0)),
        ],
        out_shape=(
            jax.ShapeDtypeStruct((n_tok, _NUM_CODES), jnp.float32),
            jax.ShapeDtypeStruct((n_tok, _NUM_CODES), jnp.float32),
            jax.ShapeDtypeStruct((n_tok, 1), jnp.int32),
        ),
    )(flat, flat_sq, emb_t, emb_sq)

    sc_stage = functools.partial(
        pl.kernel,
        mesh=plsc.VectorSubcoreMesh(core_axis_name="c", subcore_axis_name="s"),
        out_type=(
            jax.ShapeDtypeStruct((n_tok, _DIM), jnp.float32),
            jax.ShapeDtypeStruct((_NW, 16), jnp.float32),
            jax.ShapeDtypeStruct((2, _NUM_CODES), jnp.float32),
        ),
        scratch_types=[
            pltpu.VMEM((_BW,), jnp.int32),
            pltpu.VMEM((_BW, 128), jnp.float32),
            pltpu.VMEM((_BW, _DIM), jnp.float32),
            pltpu.VMEM((_BW, _DIM), jnp.float32),
            pltpu.VMEM((16,), jnp.float32),
            pltpu.VMEM((_NUM_CODES,), jnp.float32),
            pltpu.VMEM((_BW,), jnp.float32),
            pltpu.VMEM_SHARED((_NUM_CODES,), jnp.float32),
            pltpu.SemaphoreType.DMA,
        ],
    )(_sc_stage_body)
    emb_pad = jnp.pad(emb_weight, ((0, 0), (0, 128 - _DIM)))
    qst, sse, hist = sc_stage(flat, emb_pad, idx.reshape(-1))

    loss, perp = pl.pallas_call(
        functools.partial(_finalize_body, float(n_total)),
        grid=(1,),
        in_specs=[
            pl.BlockSpec((_NW, 16), lambda i: (0, 0)),
            pl.BlockSpec((2, _NUM_CODES), lambda i: (0, 0)),
        ],
        out_specs=[
            pl.BlockSpec((1, 1), lambda i: (0, 0)),
            pl.BlockSpec((1, 1), lambda i: (0, 0)),
        ],
        out_shape=(
            jax.ShapeDtypeStruct((1, 1), jnp.float32),
            jax.ShapeDtypeStruct((1, 1), jnp.float32),
        ),
    )(sse, hist)

    quantized_st = jnp.transpose(qst.reshape(inputs_shape), (0, 3, 1, 2))
    return (dist, quantized_st, loss.reshape(()), enc, idx,
            perp.reshape(()))


# speedup vs baseline: 2.2484x; 1.0054x over previous
import functools

import jax
import jax.numpy as jnp
from jax import lax
from jax.experimental import pallas as pl
from jax.experimental.pallas import tpu as pltpu
from jax.experimental.pallas import tpu_sc as plsc

_NUM_CODES = 8192
_DIM = 32
_COMMIT = 0.25
_TB = 256
_CH = 1024
_NW = 32
_BW = 256


def _vq_body(nsteps, n_total, f_ref, f2_ref, et_ref, e2_ref,
             dist_ref, loss_ref, enc_ref, idx_ref, perp_ref,
             sse_ref, hist_ref):
    i = pl.program_id(0)
    f = f_ref[...]
    f2 = f2_ref[...]
    mm = lax.dot_general(f, et_ref[...], (((1,), (0,)), ((), ())),
                         preferred_element_type=jnp.float32)

    ids = lax.broadcasted_iota(jnp.int32, (_TB, _CH), 1)
    run_min = None
    run_idx = None
    for c in range(_NUM_CODES // _CH):
        sl = pl.ds(c * _CH, _CH)
        dc = (f2 + e2_ref[:, sl]) - 2.0 * mm[:, c * _CH:(c + 1) * _CH]
        dist_ref[:, sl] = dc
        mc = jnp.min(dc, axis=1, keepdims=True)
        idc = jnp.min(jnp.where(dc == mc, ids, _NUM_CODES), axis=1,
                      keepdims=True) + c * _CH
        if c == 0:
            run_min, run_idx = mc, idc
        else:
            better = mc < run_min
            run_idx = jnp.where(better, idc, run_idx)
            run_min = jnp.where(better, mc, run_min)
    idx_ref[...] = run_idx

    for c in range(_NUM_CODES // _CH):
        enc_ref[:, pl.ds(c * _CH, _CH)] = (
            ids == run_idx - c * _CH).astype(jnp.float32)

    part = jnp.sum(run_min)
    ones_row = jnp.ones((1, _TB), jnp.float32)
    bh = lax.dot_general(ones_row, enc_ref[...], (((1,), (0,)), ((), ())),
                         preferred_element_type=jnp.float32)

    @pl.when(i == 0)
    def _():
        sse_ref[0, 0] = part
        hist_ref[...] = bh

    @pl.when(i > 0)
    def _():
        sse_ref[0, 0] = sse_ref[0, 0] + part
        hist_ref[...] = hist_ref[...] + bh

    @pl.when(i == nsteps - 1)
    def _():
        mse = sse_ref[0, 0] / n_total
        loss_ref[...] = jnp.reshape(mse + _COMMIT * mse, (1, 1))
        p = hist_ref[...] * (1.0 / n_total * _DIM)
        ent = jnp.sum(p * jnp.log(p + 1e-10))
        perp_ref[...] = jnp.reshape(jnp.exp(-ent), (1, 1))


def _sc_gather_body(ep_hbm, idx_hbm, qst_hbm, idx_v, rows_v, qst_v, sem):
    wid = lax.axis_index("s") * 2 + lax.axis_index("c")
    base = wid * _BW
    pltpu.sync_copy(idx_hbm.at[pl.ds(base, _BW)], idx_v)
    pltpu.async_copy(ep_hbm.at[idx_v], rows_v, sem).wait()

    def c_body(r, carry):
        for c in (0, 16):
            qst_v[r, pl.ds(c, 16)] = rows_v[r, pl.ds(c, 16)]
        return carry

    lax.fori_loop(0, _BW, c_body, 0)
    pltpu.sync_copy(qst_v, qst_hbm.at[pl.ds(base, _BW)])


def kernel(x, emb_weight):
    inputs = jnp.transpose(x, (0, 2, 3, 1))
    inputs_shape = inputs.shape
    flat = inputs.reshape(-1, _DIM)
    n_tok = flat.shape[0]
    nsteps = n_tok // _TB
    n_total = n_tok * _DIM

    flat_sq = jnp.sum(flat ** 2, axis=1, keepdims=True)
    emb_sq = jnp.sum(emb_weight ** 2, axis=1)[None, :]
    emb_t = emb_weight.T

    dist, loss, enc, idx, perp = pl.pallas_call(
        functools.partial(_vq_body, nsteps, float(n_total)),
        grid=(nsteps,),
        in_specs=[
            pl.BlockSpec((_TB, _DIM), lambda i: (i, 0)),
            pl.BlockSpec((_TB, 1), lambda i: (i, 0)),
            pl.BlockSpec((_DIM, _NUM_CODES), lambda i: (0, 0)),
            pl.BlockSpec((1, _NUM_CODES), lambda i: (0, 0)),
        ],
        out_specs=[
            pl.BlockSpec((_TB, _NUM_CODES), lambda i: (i, 0)),
            pl.BlockSpec((1, 1), lambda i: (0, 0)),
            pl.BlockSpec((_TB, _NUM_CODES), lambda i: (i, 0)),
            pl.BlockSpec((_TB, 1), lambda i: (i, 0)),
            pl.BlockSpec((1, 1), lambda i: (0, 0)),
        ],
        out_shape=(
            jax.ShapeDtypeStruct((n_tok, _NUM_CODES), jnp.float32),
            jax.ShapeDtypeStruct((1, 1), jnp.float32),
            jax.ShapeDtypeStruct((n_tok, _NUM_CODES), jnp.float32),
            jax.ShapeDtypeStruct((n_tok, 1), jnp.int32),
            jax.ShapeDtypeStruct((1, 1), jnp.float32),
        ),
        scratch_shapes=[
            pltpu.SMEM((1, 1), jnp.float32),
            pltpu.VMEM((1, _NUM_CODES), jnp.float32),
        ],
    )(flat, flat_sq, emb_t, emb_sq)

    sc_gather = functools.partial(
        pl.kernel,
        mesh=plsc.VectorSubcoreMesh(core_axis_name="c", subcore_axis_name="s"),
        out_type=jax.ShapeDtypeStruct((n_tok, _DIM), jnp.float32),
        scratch_types=[
            pltpu.VMEM((_BW,), jnp.int32),
            pltpu.VMEM((_BW, 128), jnp.float32),
            pltpu.VMEM((_BW, _DIM), jnp.float32),
            pltpu.SemaphoreType.DMA,
        ],
    )(_sc_gather_body)
    emb_pad = jnp.pad(emb_weight, ((0, 0), (0, 128 - _DIM)))
    qst = sc_gather(emb_pad, idx.reshape(-1))

    quantized_st = jnp.transpose(qst.reshape(inputs_shape), (0, 3, 1, 2))
    return (dist, quantized_st, loss.reshape(()), enc, idx,
            perp.reshape(()))


# speedup vs baseline: 2.2977x; 1.0219x over previous
import functools

import jax
import jax.numpy as jnp
from jax import lax
from jax.experimental import pallas as pl
from jax.experimental.pallas import tpu as pltpu

_NUM_CODES = 8192
_DIM = 32
_COMMIT = 0.25
_TB = 256
_CH = 1024


def _vq_body(nsteps, n_total, f_ref, f2_ref, e_ref, et_ref, e2_ref,
             dist_ref, qst_ref, loss_ref, enc_ref, idx_ref, perp_ref,
             sse_ref, hist_ref):
    i = pl.program_id(0)
    f = f_ref[...]
    f2 = f2_ref[...]
    mm = lax.dot_general(f, et_ref[...], (((1,), (0,)), ((), ())),
                         preferred_element_type=jnp.float32)

    ids = lax.broadcasted_iota(jnp.int32, (_TB, _CH), 1)
    run_min = None
    run_idx = None
    for c in range(_NUM_CODES // _CH):
        sl = pl.ds(c * _CH, _CH)
        dc = (f2 + e2_ref[:, sl]) - 2.0 * mm[:, c * _CH:(c + 1) * _CH]
        dist_ref[:, sl] = dc
        mc = jnp.min(dc, axis=1, keepdims=True)
        idc = jnp.min(jnp.where(dc == mc, ids, _NUM_CODES), axis=1,
                      keepdims=True) + c * _CH
        if c == 0:
            run_min, run_idx = mc, idc
        else:
            better = mc < run_min
            run_idx = jnp.where(better, idc, run_idx)
            run_min = jnp.where(better, mc, run_min)
    idx_ref[...] = run_idx

    for c in range(_NUM_CODES // _CH):
        enc_ref[:, pl.ds(c * _CH, _CH)] = (
            ids == run_idx - c * _CH).astype(jnp.float32)

    enc = enc_ref[...]
    q = lax.dot_general(enc, e_ref[...], (((1,), (0,)), ((), ())),
                        preferred_element_type=jnp.float32)
    qst_ref[...] = f + (q - f)

    part = jnp.sum(run_min)
    ones_row = jnp.ones((1, _TB), jnp.float32)
    bh = lax.dot_general(ones_row, enc, (((1,), (0,)), ((), ())),
                         preferred_element_type=jnp.float32)

    @pl.when(i == 0)
    def _():
        sse_ref[0, 0] = part
        hist_ref[...] = bh

    @pl.when(i > 0)
    def _():
        sse_ref[0, 0] = sse_ref[0, 0] + part
        hist_ref[...] = hist_ref[...] + bh

    @pl.when(i == nsteps - 1)
    def _():
        mse = sse_ref[0, 0] / n_total
        loss_ref[...] = jnp.reshape(mse + _COMMIT * mse, (1, 1))
        p = hist_ref[...] * (1.0 / n_total * _DIM)
        ent = jnp.sum(p * jnp.log(p + 1e-10))
        perp_ref[...] = jnp.reshape(jnp.exp(-ent), (1, 1))


def kernel(x, emb_weight):
    inputs = jnp.transpose(x, (0, 2, 3, 1))
    inputs_shape = inputs.shape
    flat = inputs.reshape(-1, _DIM)
    n_tok = flat.shape[0]
    nsteps = n_tok // _TB
    n_total = n_tok * _DIM

    flat_sq = jnp.sum(flat ** 2, axis=1, keepdims=True)
    emb_sq = jnp.sum(emb_weight ** 2, axis=1)[None, :]
    emb_t = emb_weight.T

    out_shapes = (
        jax.ShapeDtypeStruct((n_tok, _NUM_CODES), jnp.float32),
        jax.ShapeDtypeStruct((n_tok, _DIM), jnp.float32),
        jax.ShapeDtypeStruct((1, 1), jnp.float32),
        jax.ShapeDtypeStruct((n_tok, _NUM_CODES), jnp.float32),
        jax.ShapeDtypeStruct((n_tok, 1), jnp.int32),
        jax.ShapeDtypeStruct((1, 1), jnp.float32),
    )
    dist, qst, loss, enc, idx, perp = pl.pallas_call(
        functools.partial(_vq_body, nsteps, float(n_total)),
        grid=(nsteps,),
        in_specs=[
            pl.BlockSpec((_TB, _DIM), lambda i: (i, 0)),
            pl.BlockSpec((_TB, 1), lambda i: (i, 0)),
            pl.BlockSpec((_NUM_CODES, _DIM), lambda i: (0, 0)),
            pl.BlockSpec((_DIM, _NUM_CODES), lambda i: (0, 0)),
            pl.BlockSpec((1, _NUM_CODES), lambda i: (0, 0)),
        ],
        out_specs=[
            pl.BlockSpec((_TB, _NUM_CODES), lambda i: (i, 0)),
            pl.BlockSpec((_TB, _DIM), lambda i: (i, 0)),
            pl.BlockSpec((1, 1), lambda i: (0, 0)),
            pl.BlockSpec((_TB, _NUM_CODES), lambda i: (i, 0)),
            pl.BlockSpec((_TB, 1), lambda i: (i, 0)),
            pl.BlockSpec((1, 1), lambda i: (0, 0)),
        ],
        out_shape=out_shapes,
        scratch_shapes=[
            pltpu.SMEM((1, 1), jnp.float32),
            pltpu.VMEM((1, _NUM_CODES), jnp.float32),
        ],
    )(flat, flat_sq, emb_weight, emb_t, emb_sq)

    quantized_st = jnp.transpose(qst.reshape(inputs_shape), (0, 3, 1, 2))
    return (dist, quantized_st, loss.reshape(()), enc, idx,
            perp.reshape(()))
